# Initial kernel scaffold; baseline (speedup 1.0000x reference)
import jax
import jax.numpy as jnp
from jax.experimental import pallas as pl


def kernel(x, y):
    raise NotImplementedError("write your pallas kernel here")


# speedup vs baseline: 38.0954x; 38.0954x over previous
import functools

import jax
import jax.numpy as jnp
import numpy as np
from jax.experimental import pallas as pl

P_MIN = 0.8
THRESHOLD = 1e-10
TN = 256


def _row_loss_kernel(a_ref, c_ref, o_ref, *, m, log_ratio):
    a = a_ref[0]
    c = c_ref[0]
    d2 = (
        (a[:, 0:1] - c[0:1, :]) ** 2
        + (a[:, 1:2] - c[1:2, :]) ** 2
        + (a[:, 2:3] - c[2:3, :]) ** 2
    )
    d = jnp.maximum(jnp.sqrt(jnp.maximum(d2, 1e-12)), 1e-6)
    d1 = jnp.min(d, axis=1, keepdims=True)
    col = jax.lax.broadcasted_iota(jnp.int32, d.shape, 1)
    first = jnp.min(jnp.where(d == d1, col, m), axis=1, keepdims=True)
    dm = jnp.where(col == first, jnp.inf, d)
    d2nd = jnp.min(dm, axis=1, keepdims=True)
    t = jnp.maximum((d2nd - d1) * (1.0 / log_ratio), 1e-6)
    e = jnp.exp((d1 - d) / t)
    esum = jnp.sum(e, axis=1, keepdims=True)
    p = e / esum
    p = jnp.where(p < THRESHOLD, 0.0, p)
    o_ref[0] = jnp.sum(p * d, keepdims=True).reshape(1, 1)


def kernel(x, y):
    b, n, _ = x.shape
    m = y.shape[1]
    a = jnp.concatenate([x, y], axis=0)
    c = jnp.concatenate([y, x], axis=0).transpose(0, 2, 1)
    log_ratio = float(np.log(P_MIN * (m - 1) / (1.0 - P_MIN)))
    nt = n // TN
    grid = (2 * b, nt)
    partial = pl.pallas_call(
        functools.partial(_row_loss_kernel, m=m, log_ratio=log_ratio),
        grid=grid,
        in_specs=[
            pl.BlockSpec((1, TN, 3), lambda i, j: (i, j, 0)),
            pl.BlockSpec((1, 3, m), lambda i, j: (i, 0, 0)),
        ],
        out_specs=pl.BlockSpec((1, 1, 1), lambda i, j: (i * nt + j, 0, 0)),
        out_shape=jax.ShapeDtypeStruct((2 * b * nt, 1, 1), jnp.float32),
    )(a, c)
    return jnp.sum(partial)
